# baseline (speedup 1.0000x reference)
import functools

import jax
import jax.numpy as jnp
import numpy as np
from jax import lax
from jax.experimental import pallas as pl
from jax.experimental.pallas import tpu as pltpu
from jax.experimental.pallas import tpu_sc as plsc

_NC = 2
_NS = 16
_NW = _NC * _NS
_IDX_W = 128
_STEPS_PER_CHUNK = 4
_FMG = 16
_NSLICE = 2
_EMB_D = 32


def _pad_transpose(table_t, v, d, cb=8192):
    def body(t_ref, out_ref):
        x = t_ref[...]
        xt = x.T
        z = jnp.zeros((cb, 128 - d), jnp.float32)
        out_ref[...] = jnp.concatenate([xt, z], axis=1).astype(jnp.bfloat16)

    return pl.pallas_call(
        body,
        grid=(pl.cdiv(v, cb),),
        in_specs=[pl.BlockSpec((d, cb), lambda i: (0, i))],
        out_specs=pl.BlockSpec((cb, 128), lambda i: (i, 0)),
        out_shape=jax.ShapeDtypeStruct((v, 128), jnp.bfloat16),
    )(table_t)


def _sc_gather(idx3, idxh3, emb_table, fm_view):
    nw, steps, iw = idx3.shape
    v, dpad = emb_table.shape
    d = _EMB_D
    per_w = steps * iw
    total = nw * per_w
    spc = next(c for c in (2, 1) if steps % c == 0)
    nchunks = steps // spc
    chunk_rows = spc * iw

    mesh = plsc.VectorSubcoreMesh(
        core_axis_name="c", subcore_axis_name="s",
        num_cores=_NC, num_subcores=_NS)

    @functools.partial(
        pl.kernel,
        out_type=(
            jax.ShapeDtypeStruct((total, d), jnp.bfloat16),
            jax.ShapeDtypeStruct((total, _FMG), jnp.float32),
        ),
        mesh=mesh,
        compiler_params=pltpu.CompilerParams(use_tc_tiling_on_sc=False),
        scratch_types=[
            pltpu.VMEM((steps, iw), jnp.int32),
            pltpu.VMEM((steps, iw), jnp.int32),
            pltpu.VMEM((chunk_rows, dpad), jnp.bfloat16),
            pltpu.VMEM((chunk_rows, _FMG), jnp.float32),
            pltpu.SemaphoreType.DMA,
        ],
    )
    def sc_kernel(idx_hbm, idxh_hbm, emb_hbm, fmv_hbm, emb_out, fmw_out,
                  idx_v, idxh_v, rows_v, fmw_v, sem):
        wid = lax.axis_index("s") * _NC + lax.axis_index("c")
        base = wid * per_w
        pltpu.sync_copy(idx_hbm.at[wid], idx_v)
        pltpu.sync_copy(idxh_hbm.at[wid], idxh_v)

        def chunk_body(c, carry):
            handles = []
            for j in range(spc):
                k = c * spc + j
                handles.append(pltpu.async_copy(
                    emb_hbm.at[idx_v.at[k]],
                    rows_v.at[pl.ds(j * iw, iw)], sem))
                handles.append(pltpu.async_copy(
                    fmv_hbm.at[idxh_v.at[k]],
                    fmw_v.at[pl.ds(j * iw, iw)], sem))
            for h in handles:
                h.wait()
            row0 = base + c * chunk_rows
            pltpu.sync_copy(rows_v.at[pl.ds(0, chunk_rows), pl.ds(0, d)],
                            emb_out.at[pl.ds(row0, chunk_rows)])
            pltpu.sync_copy(fmw_v, fmw_out.at[pl.ds(row0, chunk_rows)])
            return carry

        lax.fori_loop(0, nchunks, chunk_body, 0)

    return sc_kernel(idx3, idxh3, emb_table, fm_view)


def _tc_body(emb_ref, vals_ref, fmw_ref, idx_ref, r_ref, s_ref, r16_ref,
             w1_ref, b1_ref, w2_ref, b2_ref, w3_ref, bias_ref, out_ref):
    vals = vals_ref[...]
    vexp = jnp.dot(vals, r_ref[...], preferred_element_type=jnp.float32)
    scaled = emb_ref[...].astype(jnp.float32) * vexp
    h = jnp.dot(scaled, w1_ref[...], preferred_element_type=jnp.float32)
    h = jnp.maximum(h + b1_ref[...], 0.0)
    h = jnp.dot(h, w2_ref[...], preferred_element_type=jnp.float32)
    h = jnp.maximum(h + b2_ref[...], 0.0)
    dnn = jnp.dot(h, w3_ref[...], preferred_element_type=jnp.float32)
    s16 = s_ref[...]
    sum_emb = jnp.dot(scaled, s16, preferred_element_type=jnp.float32)
    sq_sum = jnp.dot(scaled * scaled, s16, preferred_element_type=jnp.float32)
    fm2 = 0.5 * jnp.sum(sum_emb * sum_emb - sq_sum, axis=1, keepdims=True)
    tb, fg = fmw_ref.shape
    sel = (idx_ref[...] & (_FMG - 1)).astype(jnp.float32)
    sel_exp = jnp.dot(sel, r16_ref[...], preferred_element_type=jnp.float32)
    vexp16 = jnp.dot(vals, r16_ref[...], preferred_element_type=jnp.float32)
    lane = (lax.broadcasted_iota(jnp.int32, (tb, fg), 1)
            & (_FMG - 1)).astype(jnp.float32)
    fmval = jnp.where(sel_exp == lane, fmw_ref[...], 0.0)
    fm1 = jnp.sum(fmval * vexp16, axis=1, keepdims=True)
    z = dnn + fm2 + fm1 + bias_ref[...]
    out_ref[...] = 1.0 / (1.0 + jnp.exp(-z))


def _tc_forward(emb2, vals, fmw16, idxs32, r_mat, s_mat, r16_mat,
                W1, b1, W2, b2, W3, bias, tb=512, interpret=False):
    b, fd = emb2.shape
    f = vals.shape[1]
    d = s_mat.shape[1]
    fg = f * _FMG
    h1 = W1.shape[1]
    h2 = W2.shape[1]
    grid = (b // tb,)
    return pl.pallas_call(
        _tc_body,
        grid=grid,
        in_specs=[
            pl.BlockSpec((tb, fd), lambda i: (i, 0)),
            pl.BlockSpec((tb, f), lambda i: (i, 0)),
            pl.BlockSpec((tb, fg), lambda i: (i, 0)),
            pl.BlockSpec((tb, f), lambda i: (i, 0)),
            pl.BlockSpec((f, fd), lambda i: (0, 0)),
            pl.BlockSpec((fd, d), lambda i: (0, 0)),
            pl.BlockSpec((f, fg), lambda i: (0, 0)),
            pl.BlockSpec((fd, h1), lambda i: (0, 0)),
            pl.BlockSpec((1, h1), lambda i: (0, 0)),
            pl.BlockSpec((h1, h2), lambda i: (0, 0)),
            pl.BlockSpec((1, h2), lambda i: (0, 0)),
            pl.BlockSpec((h2, 1), lambda i: (0, 0)),
            pl.BlockSpec((1, 1), lambda i: (0, 0)),
        ],
        out_specs=pl.BlockSpec((tb, 1), lambda i: (i, 0)),
        out_shape=jax.ShapeDtypeStruct((b, 1), jnp.float32),
        interpret=interpret,
    )(emb2, vals, fmw16, idxs32, r_mat, s_mat, r16_mat,
      W1, b1, W2, b2, W3, bias)


def kernel(idxs, vals, shared_emb_table, fm_first_table, fm_bias,
           W1, b1, W2, b2, W3, b3):
    b, f = idxs.shape
    v, d = shared_emb_table.shape
    idxs32 = idxs.astype(jnp.int32)
    fm_view = fm_first_table.reshape(v // _FMG, _FMG)
    emb_lin = _pad_transpose(shared_emb_table.T, v, d)

    r_mat = jnp.asarray(
        np.kron(np.eye(f, dtype=np.float32), np.ones((1, d), np.float32)))
    s_mat = jnp.asarray(np.tile(np.eye(d, dtype=np.float32), (f, 1)))
    r16_mat = jnp.asarray(
        np.kron(np.eye(f, dtype=np.float32), np.ones((1, _FMG), np.float32)))
    bias = (b3 + fm_bias).reshape(1, 1).astype(jnp.float32)
    b1r = b1.reshape(1, -1)
    b2r = b2.reshape(1, -1)

    bs = b // _NSLICE
    outs = []
    for s in range(_NSLICE):
        sl = slice(s * bs, (s + 1) * bs)
        idxs_s = idxs32[sl]
        total = bs * f
        per_w = total // _NW
        steps = per_w // _IDX_W
        idx3 = idxs_s.reshape(_NW, steps, _IDX_W)
        idxh3 = (idxs_s >> 4).reshape(_NW, steps, _IDX_W)
        emb_flat, fmw16_flat = _sc_gather(idx3, idxh3, emb_lin, fm_view)
        outs.append(_tc_forward(
            emb_flat.reshape(bs, f * d), vals[sl],
            fmw16_flat.reshape(bs, f * _FMG), idxs_s,
            r_mat, s_mat, r16_mat, W1, b1r, W2, b2r, W3, bias))
    return jnp.concatenate(outs, axis=0)


# speedup vs baseline: 2.3001x; 2.3001x over previous
import functools

import jax
import jax.numpy as jnp
import numpy as np
from jax import lax
from jax.experimental import pallas as pl
from jax.experimental.pallas import tpu as pltpu
from jax.experimental.pallas import tpu_sc as plsc

_NC = 2
_NS = 16
_NW = _NC * _NS
_IDX_W = 128
_STEPS_PER_CHUNK = 4
_FMG = 16
_NSLICE = 2
_EMB_D = 32


def _pad_transpose(table_t, v, d, cb=16384):
    def body(t_ref, out_ref):
        x = t_ref[...]
        xt = x.T
        z = jnp.zeros((cb, 128 - d), jnp.float32)
        out_ref[...] = jnp.concatenate([xt, z], axis=1)

    return pl.pallas_call(
        body,
        grid=(pl.cdiv(v, cb),),
        in_specs=[pl.BlockSpec((d, cb), lambda i: (0, i))],
        out_specs=pl.BlockSpec((cb, 128), lambda i: (i, 0)),
        out_shape=jax.ShapeDtypeStruct((v, 128), jnp.float32),
    )(table_t)


def _sc_gather(idx3, idxh3, emb_table, fm_view):
    nw, steps, iw = idx3.shape
    v, dpad = emb_table.shape
    d = _EMB_D
    per_w = steps * iw
    total = nw * per_w
    spc = next(c for c in (2, 1) if steps % c == 0)
    nchunks = steps // spc
    chunk_rows = spc * iw

    mesh = plsc.VectorSubcoreMesh(
        core_axis_name="c", subcore_axis_name="s",
        num_cores=_NC, num_subcores=_NS)

    @functools.partial(
        pl.kernel,
        out_type=(
            jax.ShapeDtypeStruct((total, d), jnp.float32),
            jax.ShapeDtypeStruct((total, _FMG), jnp.float32),
        ),
        mesh=mesh,
        compiler_params=pltpu.CompilerParams(use_tc_tiling_on_sc=False),
        scratch_types=[
            pltpu.VMEM((steps, iw), jnp.int32),
            pltpu.VMEM((steps, iw), jnp.int32),
            pltpu.VMEM((chunk_rows, dpad), jnp.float32),
            pltpu.VMEM((chunk_rows, _FMG), jnp.float32),
            pltpu.SemaphoreType.DMA,
        ],
    )
    def sc_kernel(idx_hbm, idxh_hbm, emb_hbm, fmv_hbm, emb_out, fmw_out,
                  idx_v, idxh_v, rows_v, fmw_v, sem):
        wid = lax.axis_index("s") * _NC + lax.axis_index("c")
        base = wid * per_w
        pltpu.sync_copy(idx_hbm.at[wid], idx_v)
        pltpu.sync_copy(idxh_hbm.at[wid], idxh_v)

        def chunk_body(c, carry):
            handles = []
            for j in range(spc):
                k = c * spc + j
                handles.append(pltpu.async_copy(
                    emb_hbm.at[idx_v.at[k]],
                    rows_v.at[pl.ds(j * iw, iw)], sem))
                handles.append(pltpu.async_copy(
                    fmv_hbm.at[idxh_v.at[k]],
                    fmw_v.at[pl.ds(j * iw, iw)], sem))
            for h in handles:
                h.wait()
            row0 = base + c * chunk_rows
            pltpu.sync_copy(rows_v.at[pl.ds(0, chunk_rows), pl.ds(0, d)],
                            emb_out.at[pl.ds(row0, chunk_rows)])
            pltpu.sync_copy(fmw_v, fmw_out.at[pl.ds(row0, chunk_rows)])
            return carry

        lax.fori_loop(0, nchunks, chunk_body, 0)

    return sc_kernel(idx3, idxh3, emb_table, fm_view)


def _tc_body(emb_ref, vals_ref, fmw_ref, idx_ref, r_ref, s_ref, r16_ref,
             w1_ref, b1_ref, w2_ref, b2_ref, w3_ref, bias_ref, out_ref):
    vals = vals_ref[...]
    vexp = jnp.dot(vals, r_ref[...], preferred_element_type=jnp.float32)
    scaled = emb_ref[...] * vexp
    h = jnp.dot(scaled, w1_ref[...], preferred_element_type=jnp.float32)
    h = jnp.maximum(h + b1_ref[...], 0.0)
    h = jnp.dot(h, w2_ref[...], preferred_element_type=jnp.float32)
    h = jnp.maximum(h + b2_ref[...], 0.0)
    dnn = jnp.dot(h, w3_ref[...], preferred_element_type=jnp.float32)
    s16 = s_ref[...]
    sum_emb = jnp.dot(scaled, s16, preferred_element_type=jnp.float32)
    sq_sum = jnp.dot(scaled * scaled, s16, preferred_element_type=jnp.float32)
    fm2 = 0.5 * jnp.sum(sum_emb * sum_emb - sq_sum, axis=1, keepdims=True)
    tb, fg = fmw_ref.shape
    sel = (idx_ref[...] & (_FMG - 1)).astype(jnp.float32)
    sel_exp = jnp.dot(sel, r16_ref[...], preferred_element_type=jnp.float32)
    vexp16 = jnp.dot(vals, r16_ref[...], preferred_element_type=jnp.float32)
    lane = (lax.broadcasted_iota(jnp.int32, (tb, fg), 1)
            & (_FMG - 1)).astype(jnp.float32)
    fmval = jnp.where(sel_exp == lane, fmw_ref[...], 0.0)
    fm1 = jnp.sum(fmval * vexp16, axis=1, keepdims=True)
    z = dnn + fm2 + fm1 + bias_ref[...]
    out_ref[...] = 1.0 / (1.0 + jnp.exp(-z))


def _tc_forward(emb2, vals, fmw16, idxs32, r_mat, s_mat, r16_mat,
                W1, b1, W2, b2, W3, bias, tb=512, interpret=False):
    b, fd = emb2.shape
    f = vals.shape[1]
    d = s_mat.shape[1]
    fg = f * _FMG
    h1 = W1.shape[1]
    h2 = W2.shape[1]
    grid = (b // tb,)
    return pl.pallas_call(
        _tc_body,
        grid=grid,
        in_specs=[
            pl.BlockSpec((tb, fd), lambda i: (i, 0)),
            pl.BlockSpec((tb, f), lambda i: (i, 0)),
            pl.BlockSpec((tb, fg), lambda i: (i, 0)),
            pl.BlockSpec((tb, f), lambda i: (i, 0)),
            pl.BlockSpec((f, fd), lambda i: (0, 0)),
            pl.BlockSpec((fd, d), lambda i: (0, 0)),
            pl.BlockSpec((f, fg), lambda i: (0, 0)),
            pl.BlockSpec((fd, h1), lambda i: (0, 0)),
            pl.BlockSpec((1, h1), lambda i: (0, 0)),
            pl.BlockSpec((h1, h2), lambda i: (0, 0)),
            pl.BlockSpec((1, h2), lambda i: (0, 0)),
            pl.BlockSpec((h2, 1), lambda i: (0, 0)),
            pl.BlockSpec((1, 1), lambda i: (0, 0)),
        ],
        out_specs=pl.BlockSpec((tb, 1), lambda i: (i, 0)),
        out_shape=jax.ShapeDtypeStruct((b, 1), jnp.float32),
        interpret=interpret,
    )(emb2, vals, fmw16, idxs32, r_mat, s_mat, r16_mat,
      W1, b1, W2, b2, W3, bias)


def kernel(idxs, vals, shared_emb_table, fm_first_table, fm_bias,
           W1, b1, W2, b2, W3, b3):
    b, f = idxs.shape
    v, d = shared_emb_table.shape
    idxs32 = idxs.astype(jnp.int32)
    fm_view = fm_first_table.reshape(v // _FMG, _FMG)
    emb_lin = _pad_transpose(shared_emb_table.T, v, d)

    r_mat = jnp.asarray(
        np.kron(np.eye(f, dtype=np.float32), np.ones((1, d), np.float32)))
    s_mat = jnp.asarray(np.tile(np.eye(d, dtype=np.float32), (f, 1)))
    r16_mat = jnp.asarray(
        np.kron(np.eye(f, dtype=np.float32), np.ones((1, _FMG), np.float32)))
    bias = (b3 + fm_bias).reshape(1, 1).astype(jnp.float32)
    b1r = b1.reshape(1, -1)
    b2r = b2.reshape(1, -1)

    bs = b // _NSLICE
    outs = []
    for s in range(_NSLICE):
        sl = slice(s * bs, (s + 1) * bs)
        idxs_s = idxs32[sl]
        total = bs * f
        per_w = total // _NW
        steps = per_w // _IDX_W
        idx3 = idxs_s.reshape(_NW, steps, _IDX_W)
        idxh3 = (idxs_s >> 4).reshape(_NW, steps, _IDX_W)
        emb_flat, fmw16_flat = _sc_gather(idx3, idxh3, emb_lin, fm_view)
        outs.append(_tc_forward(
            emb_flat.reshape(bs, f * d), vals[sl],
            fmw16_flat.reshape(bs, f * _FMG), idxs_s,
            r_mat, s_mat, r16_mat, W1, b1r, W2, b2r, W3, bias))
    return jnp.concatenate(outs, axis=0)


# speedup vs baseline: 2.3149x; 1.0064x over previous
import functools

import jax
import jax.numpy as jnp
import numpy as np
from jax import lax
from jax.experimental import pallas as pl
from jax.experimental.pallas import tpu as pltpu
from jax.experimental.pallas import tpu_sc as plsc

_NC = 2
_NS = 16
_NW = _NC * _NS
_IDX_W = 128
_STEPS_PER_CHUNK = 4
_FMG = 16
_NSLICE = 2
_EMB_D = 32


def _pad_transpose(table_t, v, d, cb=32768):
    def body(t_ref, out_ref):
        x = t_ref[...]
        xt = x.T
        z = jnp.zeros((cb, 128 - d), jnp.float32)
        out_ref[...] = jnp.concatenate([xt, z], axis=1)

    return pl.pallas_call(
        body,
        grid=(pl.cdiv(v, cb),),
        in_specs=[pl.BlockSpec((d, cb), lambda i: (0, i))],
        out_specs=pl.BlockSpec((cb, 128), lambda i: (i, 0)),
        out_shape=jax.ShapeDtypeStruct((v, 128), jnp.float32),
    )(table_t)


def _sc_gather(idx3, idxh3, emb_table, fm_view):
    nw, steps, iw = idx3.shape
    v, dpad = emb_table.shape
    d = _EMB_D
    per_w = steps * iw
    total = nw * per_w
    spc = next(c for c in (2, 1) if steps % c == 0)
    nchunks = steps // spc
    chunk_rows = spc * iw

    mesh = plsc.VectorSubcoreMesh(
        core_axis_name="c", subcore_axis_name="s",
        num_cores=_NC, num_subcores=_NS)

    @functools.partial(
        pl.kernel,
        out_type=(
            jax.ShapeDtypeStruct((total, d), jnp.float32),
            jax.ShapeDtypeStruct((total, _FMG), jnp.float32),
        ),
        mesh=mesh,
        compiler_params=pltpu.CompilerParams(use_tc_tiling_on_sc=False),
        scratch_types=[
            pltpu.VMEM((steps, iw), jnp.int32),
            pltpu.VMEM((steps, iw), jnp.int32),
            pltpu.VMEM((chunk_rows, dpad), jnp.float32),
            pltpu.VMEM((chunk_rows, _FMG), jnp.float32),
            pltpu.SemaphoreType.DMA,
        ],
    )
    def sc_kernel(idx_hbm, idxh_hbm, emb_hbm, fmv_hbm, emb_out, fmw_out,
                  idx_v, idxh_v, rows_v, fmw_v, sem):
        wid = lax.axis_index("s") * _NC + lax.axis_index("c")
        base = wid * per_w
        pltpu.sync_copy(idx_hbm.at[wid], idx_v)
        pltpu.sync_copy(idxh_hbm.at[wid], idxh_v)

        def chunk_body(c, carry):
            handles = []
            for j in range(spc):
                k = c * spc + j
                handles.append(pltpu.async_copy(
                    emb_hbm.at[idx_v.at[k]],
                    rows_v.at[pl.ds(j * iw, iw)], sem))
                handles.append(pltpu.async_copy(
                    fmv_hbm.at[idxh_v.at[k]],
                    fmw_v.at[pl.ds(j * iw, iw)], sem))
            for h in handles:
                h.wait()
            row0 = base + c * chunk_rows
            pltpu.sync_copy(rows_v.at[pl.ds(0, chunk_rows), pl.ds(0, d)],
                            emb_out.at[pl.ds(row0, chunk_rows)])
            pltpu.sync_copy(fmw_v, fmw_out.at[pl.ds(row0, chunk_rows)])
            return carry

        lax.fori_loop(0, nchunks, chunk_body, 0)

    return sc_kernel(idx3, idxh3, emb_table, fm_view)


def _tc_body(emb_ref, vals_ref, fmw_ref, idx_ref, r_ref, s_ref, r16_ref,
             w1_ref, b1_ref, w2_ref, b2_ref, w3_ref, bias_ref, out_ref):
    vals = vals_ref[...]
    vexp = jnp.dot(vals, r_ref[...], preferred_element_type=jnp.float32)
    scaled = emb_ref[...] * vexp
    h = jnp.dot(scaled, w1_ref[...], preferred_element_type=jnp.float32)
    h = jnp.maximum(h + b1_ref[...], 0.0)
    h = jnp.dot(h, w2_ref[...], preferred_element_type=jnp.float32)
    h = jnp.maximum(h + b2_ref[...], 0.0)
    dnn = jnp.dot(h, w3_ref[...], preferred_element_type=jnp.float32)
    s16 = s_ref[...]
    sum_emb = jnp.dot(scaled, s16, preferred_element_type=jnp.float32)
    sq_sum = jnp.dot(scaled * scaled, s16, preferred_element_type=jnp.float32)
    fm2 = 0.5 * jnp.sum(sum_emb * sum_emb - sq_sum, axis=1, keepdims=True)
    tb, fg = fmw_ref.shape
    sel = (idx_ref[...] & (_FMG - 1)).astype(jnp.float32)
    sel_exp = jnp.dot(sel, r16_ref[...], preferred_element_type=jnp.float32)
    vexp16 = jnp.dot(vals, r16_ref[...], preferred_element_type=jnp.float32)
    lane = (lax.broadcasted_iota(jnp.int32, (tb, fg), 1)
            & (_FMG - 1)).astype(jnp.float32)
    fmval = jnp.where(sel_exp == lane, fmw_ref[...], 0.0)
    fm1 = jnp.sum(fmval * vexp16, axis=1, keepdims=True)
    z = dnn + fm2 + fm1 + bias_ref[...]
    out_ref[...] = 1.0 / (1.0 + jnp.exp(-z))


def _tc_forward(emb2, vals, fmw16, idxs32, r_mat, s_mat, r16_mat,
                W1, b1, W2, b2, W3, bias, tb=512, interpret=False):
    b, fd = emb2.shape
    f = vals.shape[1]
    d = s_mat.shape[1]
    fg = f * _FMG
    h1 = W1.shape[1]
    h2 = W2.shape[1]
    grid = (b // tb,)
    return pl.pallas_call(
        _tc_body,
        grid=grid,
        in_specs=[
            pl.BlockSpec((tb, fd), lambda i: (i, 0)),
            pl.BlockSpec((tb, f), lambda i: (i, 0)),
            pl.BlockSpec((tb, fg), lambda i: (i, 0)),
            pl.BlockSpec((tb, f), lambda i: (i, 0)),
            pl.BlockSpec((f, fd), lambda i: (0, 0)),
            pl.BlockSpec((fd, d), lambda i: (0, 0)),
            pl.BlockSpec((f, fg), lambda i: (0, 0)),
            pl.BlockSpec((fd, h1), lambda i: (0, 0)),
            pl.BlockSpec((1, h1), lambda i: (0, 0)),
            pl.BlockSpec((h1, h2), lambda i: (0, 0)),
            pl.BlockSpec((1, h2), lambda i: (0, 0)),
            pl.BlockSpec((h2, 1), lambda i: (0, 0)),
            pl.BlockSpec((1, 1), lambda i: (0, 0)),
        ],
        out_specs=pl.BlockSpec((tb, 1), lambda i: (i, 0)),
        out_shape=jax.ShapeDtypeStruct((b, 1), jnp.float32),
        interpret=interpret,
    )(emb2, vals, fmw16, idxs32, r_mat, s_mat, r16_mat,
      W1, b1, W2, b2, W3, bias)


def kernel(idxs, vals, shared_emb_table, fm_first_table, fm_bias,
           W1, b1, W2, b2, W3, b3):
    b, f = idxs.shape
    v, d = shared_emb_table.shape
    idxs32 = idxs.astype(jnp.int32)
    fm_view = fm_first_table.reshape(v // _FMG, _FMG)
    emb_lin = _pad_transpose(shared_emb_table.T, v, d)

    r_mat = jnp.asarray(
        np.kron(np.eye(f, dtype=np.float32), np.ones((1, d), np.float32)))
    s_mat = jnp.asarray(np.tile(np.eye(d, dtype=np.float32), (f, 1)))
    r16_mat = jnp.asarray(
        np.kron(np.eye(f, dtype=np.float32), np.ones((1, _FMG), np.float32)))
    bias = (b3 + fm_bias).reshape(1, 1).astype(jnp.float32)
    b1r = b1.reshape(1, -1)
    b2r = b2.reshape(1, -1)

    bs = b // _NSLICE
    outs = []
    for s in range(_NSLICE):
        sl = slice(s * bs, (s + 1) * bs)
        idxs_s = idxs32[sl]
        total = bs * f
        per_w = total // _NW
        steps = per_w // _IDX_W
        idx3 = idxs_s.reshape(_NW, steps, _IDX_W)
        idxh3 = (idxs_s >> 4).reshape(_NW, steps, _IDX_W)
        emb_flat, fmw16_flat = _sc_gather(idx3, idxh3, emb_lin, fm_view)
        outs.append(_tc_forward(
            emb_flat.reshape(bs, f * d), vals[sl],
            fmw16_flat.reshape(bs, f * _FMG), idxs_s,
            r_mat, s_mat, r16_mat, W1, b1r, W2, b2r, W3, bias))
    return jnp.concatenate(outs, axis=0)
